# Initial kernel scaffold; baseline (speedup 1.0000x reference)
import jax
import jax.numpy as jnp
from jax.experimental import pallas as pl


def kernel(xyz, features, t_emb, condition_emb, params):
    raise NotImplementedError("write your pallas kernel here")


# speedup vs baseline: 5.7494x; 5.7494x over previous
import jax
import jax.numpy as jnp
from jax.experimental import pallas as pl
from functools import partial

_B, _N, _S, _K = 4, 4096, 1024, 32
_R2 = 0.2 ** 2
_CF, _CO = 128, 256
_EPS = 1e-5
_NG = 32
_GS = _CO // _NG
_SCALE = _CO ** -0.5

_f32 = jnp.float32
_i32 = jnp.int32


def _iota(shape, dim):
    return jax.lax.broadcasted_iota(_i32, shape, dim)


def _sel_mats():
    sel = (_iota((_NG, _CO), 1) // _GS == _iota((_NG, _CO), 0)).astype(_f32)
    selC = (_iota((_CO, _NG), 0) // _GS == _iota((_CO, _NG), 1)).astype(_f32)
    return sel, selC


def _gn(x, gamma, beta):
    M = x.shape[1]
    sel, selC = _sel_mats()
    s1 = jnp.sum(x, axis=1, keepdims=True)
    s2 = jnp.sum(x * x, axis=1, keepdims=True)
    g1 = jax.lax.dot(sel, s1)
    g2 = jax.lax.dot(sel, s2)
    cnt = _GS * M
    m = g1 / cnt
    v = g2 / cnt - m * m
    inv = jax.lax.rsqrt(v + _EPS)
    mC = jax.lax.dot(selC, m)
    invC = jax.lax.dot(selC, inv)
    return (x - mC) * invC * gamma + beta


def _relu(x):
    return jnp.maximum(x, 0.0)


def _dot_t(a, b):
    return jax.lax.dot_general(a, b, (((1,), (1,)), ((), ())),
                               preferred_element_type=_f32)


def _emb_kern(f_ref, t_ref, c_ref,
              t1w, t1b, t2w, t2b, c1w, c1b, c2w, c2b,
              f1a, f1bm, f1b_, f1g, f1be,
              f2a, f2bm, f2b_, f2g, f2be,
              rsw, rsb, rsg, rsbe, out_ref):
    f = f_ref[0]
    t = t_ref[0]
    c = c_ref[0]
    t1 = _relu(_dot_t(t, t1w[...]) + t1b[...])
    t2 = _relu(_dot_t(t1, t2w[...]) + t2b[...])
    c1 = _relu(_dot_t(c, c1w[...]) + c1b[...])
    c2 = _relu(_dot_t(c1, c2w[...]) + c2b[...])
    h = jax.lax.dot(f1a[...], f) + _dot_t(f1bm[...], t2) + f1b_[...]
    h = _relu(_gn(h, f1g[...], f1be[...]))
    h2 = jax.lax.dot(f2a[...], h) + _dot_t(f2bm[...], c2) + f2b_[...]
    h2 = _relu(_gn(h2, f2g[...], f2be[...]))
    res = jax.lax.dot(rsw[...], f) + rsb[...]
    res = _relu(_gn(res, rsg[...], rsbe[...]))
    out_ref[0] = h2 + res


def _fps_kern(xyzT_ref, xyz_ref, fembT_ref, nxyz_ref, nfeat_ref):
    x = xyzT_ref[0, 0:1, :]
    y = xyzT_ref[0, 1:2, :]
    z = xyzT_ref[0, 2:3, :]
    iota_n = _iota((1, _N), 1)
    iota_s = _iota((_S, 1), 0)

    def body(i, st):
        dists, idxs, far = st
        idxs = jnp.where(iota_s == i, far[0, 0], idxs)
        oh = (iota_n == far).astype(_f32)
        cx = jnp.sum(x * oh, axis=1, keepdims=True)
        cy = jnp.sum(y * oh, axis=1, keepdims=True)
        cz = jnp.sum(z * oh, axis=1, keepdims=True)
        d = (x - cx) ** 2 + (y - cy) ** 2 + (z - cz) ** 2
        dists = jnp.minimum(dists, d)
        mx = jnp.max(dists, axis=1, keepdims=True)
        far = jnp.min(jnp.where(dists == mx, iota_n, _N),
                      axis=1, keepdims=True).astype(_i32)
        return dists, idxs, far

    dists0 = jnp.full((1, _N), 1e10, _f32)
    idxs0 = jnp.zeros((_S, 1), _i32)
    far0 = jnp.zeros((1, 1), _i32)
    _, idxs, _ = jax.lax.fori_loop(0, _S, body, (dists0, idxs0, far0))

    fembb = fembT_ref[0]
    iota_full = _iota((256, _N), 1)
    for sc in range(4):
        idc = jax.lax.slice(idxs, (sc * 256, 0), ((sc + 1) * 256, 1))
        oh = (idc == iota_full).astype(_f32)
        gx = jnp.sum(oh * x, axis=1, keepdims=True)
        gy = jnp.sum(oh * y, axis=1, keepdims=True)
        gz = jnp.sum(oh * z, axis=1, keepdims=True)
        nxyz_ref[0, sc * 256:(sc + 1) * 256, :] = jnp.concatenate(
            [gx, gy, gz], axis=1)
        nfeat_ref[0, sc * 256:(sc + 1) * 256, :] = jax.lax.dot(oh, fembb)


_STQ = 256
_CHN = 512


def _ball_kern(xyzT_ref, nxyz_ref, gidx_ref, cnt_ref):
    nx = nxyz_ref[0]
    cx = nx[:, 0:1]
    cy = nx[:, 1:2]
    cz = nx[:, 2:3]
    px = xyzT_ref[0, 0:1, :]
    py = xyzT_ref[0, 1:2, :]
    pz = xyzT_ref[0, 2:3, :]
    d = (cx - px) ** 2 + (cy - py) ** 2 + (cz - pz) ** 2
    valid = (d < _R2).astype(_f32)
    U = (_iota((_CHN, _CHN), 0) <= _iota((_CHN, _CHN), 1)).astype(_f32)
    offs = jnp.zeros((_STQ, 1), _f32)
    ranks = []
    for ci in range(_N // _CHN):
        vc = jax.lax.slice(valid, (0, ci * _CHN), (_STQ, (ci + 1) * _CHN))
        rc = jax.lax.dot(vc, U) + offs
        ranks.append(rc)
        offs = rc[:, _CHN - 1:_CHN]
    rank = jnp.concatenate(ranks, axis=1)
    total = rank[:, _N - 1:_N]
    count = jnp.minimum(total, float(_K))
    cols = [jnp.sum((rank <= float(r)).astype(_f32), axis=1, keepdims=True)
            for r in range(_K)]
    idxs = jnp.concatenate(cols, axis=1)
    first = jnp.where(total > 0, idxs[:, 0:1], 0.0)
    rmask = _iota((_STQ, _K), 1).astype(_f32) < count
    gidx = jnp.where(rmask, idxs, first)
    gidx_ref[0] = gidx.astype(_i32)
    cnt_ref[0] = count


_ST = 128
_RW = _ST * _K
_GCH = 512


def _group_kern(fembT_ref, xyz_ref, nxyz_ref, nfeat_ref, gidx_ref, cnt_ref,
                fqw, fqb, fkg, fkf, fkb, fvg, fvf, fvb,
                attn_ref, vconv_ref, stat_ref):
    fi = gidx_ref[0, 0]
    Gf = jnp.zeros((_RW, _CO), _f32)
    Gx = jnp.zeros((_RW, 3), _f32)
    for ci in range(_N // _GCH):
        io = _iota((1, _GCH), 1) + ci * _GCH
        oh = (fi == io).astype(_f32)
        fc = jax.lax.slice(fembT_ref[0], (ci * _GCH, 0), ((ci + 1) * _GCH, _CO))
        xc = jax.lax.slice(xyz_ref[0], (ci * _GCH, 0), ((ci + 1) * _GCH, 3))
        Gf = Gf + jax.lax.dot(oh, fc)
        Gx = Gx + jax.lax.dot(oh, xc)
    Rm = (_iota((_RW, _ST), 0) // _K == _iota((_RW, _ST), 1)).astype(_f32)
    nxB = jax.lax.dot(Rm, nxyz_ref[0])
    geom = jnp.concatenate([Gx - nxB, Gx, nxB], axis=1)
    q = _dot_t(nfeat_ref[0], fqw[...]) + fqb[...]
    qB = jax.lax.dot(Rm, q)
    kvec = _dot_t(Gf, fkf[...]) + _dot_t(geom, fkg[...]) + fkb[...]
    logit = jnp.sum(qB * kvec, axis=1, keepdims=True) * _SCALE
    cnt = jnp.maximum(cnt_ref[0], 1.0)
    cntB = jax.lax.dot(Rm, cnt)
    jmod = (_iota((_RW, 1), 0) % _K).astype(_f32)
    mask = (jmod < cntB).astype(_f32)
    logit = logit * mask - 1e9 * (1.0 - mask)
    e = jnp.exp(logit)
    den = jax.lax.dot_general(Rm, e, (((0,), (0,)), ((), ())),
                              preferred_element_type=_f32)
    attn = e / jax.lax.dot(Rm, den)
    attn_ref[0, 0] = attn
    vc = _dot_t(Gf, fvf[...]) + _dot_t(geom, fvg[...]) + fvb[...]
    vconv_ref[0, 0] = vc
    sel, _ = _sel_mats()
    s1 = jnp.sum(vc, axis=0, keepdims=True)
    s2 = jnp.sum(vc * vc, axis=0, keepdims=True)
    g1 = jax.lax.dot_general(sel, s1, (((1,), (1,)), ((), ())),
                             preferred_element_type=_f32)
    g2 = jax.lax.dot_general(sel, s2, (((1,), (1,)), ((), ())),
                             preferred_element_type=_f32)
    stat_ref[0, 0] = jnp.concatenate([g1, g2], axis=1)


def _finish_kern(vconv_ref, attn_ref, stat_ref, fvgam, fvbet, out_ref):
    st = stat_ref[0]
    s = jnp.sum(st, axis=0)
    cnt = float(_GS * _S * _K)
    m = s[:, 0:1] / cnt
    v = s[:, 1:2] / cnt - m * m
    inv = jax.lax.rsqrt(v + _EPS)
    _, selC = _sel_mats()
    mC = jax.lax.dot_general(m, selC, (((0,), (1,)), ((), ())),
                             preferred_element_type=_f32)
    invC = jax.lax.dot_general(inv, selC, (((0,), (1,)), ((), ())),
                               preferred_element_type=_f32)
    vc = vconv_ref[0, 0]
    vn = _relu((vc - mC) * invC * fvgam[...] + fvbet[...])
    w = attn_ref[0, 0]
    Rm = (_iota((_RW, _ST), 0) // _K == _iota((_RW, _ST), 1)).astype(_f32)
    out = jax.lax.dot_general(Rm, w * vn, (((0,), (0,)), ((), ())),
                              preferred_element_type=_f32)
    out_ref[0] = out * float(_K)


def kernel(xyz, features, t_emb, condition_emb, params):
    p = params
    xyzT = xyz.transpose(0, 2, 1)
    col = lambda a: a.reshape(-1, 1)
    row = lambda a: a.reshape(1, -1)

    femb = pl.pallas_call(
        _emb_kern,
        grid=(_B,),
        in_specs=[
            pl.BlockSpec((1, _CF, _N), lambda b: (b, 0, 0)),
            pl.BlockSpec((1, 1, 512), lambda b: (b, 0, 0)),
            pl.BlockSpec((1, 1, 1024), lambda b: (b, 0, 0)),
        ] + [pl.BlockSpec(None)] * 22,
        out_specs=pl.BlockSpec((1, _CO, _N), lambda b: (b, 0, 0)),
        out_shape=jax.ShapeDtypeStruct((_B, _CO, _N), _f32),
    )(features, t_emb.reshape(_B, 1, 512), condition_emb.reshape(_B, 1, 1024),
      p['t1_W'], row(p['t1_b']), p['t2_W'], row(p['t2_b']),
      p['c1_W'], row(p['c1_b']), p['c2_W'], row(p['c2_b']),
      p['f1_W'][:, :_CF], p['f1_W'][:, _CF:], col(p['f1_b']),
      col(p['f1_g']), col(p['f1_be']),
      p['f2_W'][:, :_CO], p['f2_W'][:, _CO:], col(p['f2_b']),
      col(p['f2_g']), col(p['f2_be']),
      p['rs_W'], col(p['rs_b']), col(p['rs_g']), col(p['rs_be']))

    fembT = femb.transpose(0, 2, 1)

    new_xyz, new_feat = pl.pallas_call(
        _fps_kern,
        grid=(_B,),
        in_specs=[
            pl.BlockSpec((1, 3, _N), lambda b: (b, 0, 0)),
            pl.BlockSpec((1, _N, 3), lambda b: (b, 0, 0)),
            pl.BlockSpec((1, _N, _CO), lambda b: (b, 0, 0)),
        ],
        out_specs=[
            pl.BlockSpec((1, _S, 3), lambda b: (b, 0, 0)),
            pl.BlockSpec((1, _S, _CO), lambda b: (b, 0, 0)),
        ],
        out_shape=[
            jax.ShapeDtypeStruct((_B, _S, 3), _f32),
            jax.ShapeDtypeStruct((_B, _S, _CO), _f32),
        ],
    )(xyzT, xyz, fembT)

    gidx, count = pl.pallas_call(
        _ball_kern,
        grid=(_B, _S // _STQ),
        in_specs=[
            pl.BlockSpec((1, 3, _N), lambda b, s: (b, 0, 0)),
            pl.BlockSpec((1, _STQ, 3), lambda b, s: (b, s, 0)),
        ],
        out_specs=[
            pl.BlockSpec((1, _STQ, _K), lambda b, s: (b, s, 0)),
            pl.BlockSpec((1, _STQ, 1), lambda b, s: (b, s, 0)),
        ],
        out_shape=[
            jax.ShapeDtypeStruct((_B, _S, _K), _i32),
            jax.ShapeDtypeStruct((_B, _S, 1), _f32),
        ],
    )(xyzT, new_xyz)

    nst = _S // _ST
    gidxF = gidx.reshape(_B, nst, _RW, 1)

    attn, vconv, stats = pl.pallas_call(
        _group_kern,
        grid=(_B, nst),
        in_specs=[
            pl.BlockSpec((1, _N, _CO), lambda b, s: (b, 0, 0)),
            pl.BlockSpec((1, _N, 3), lambda b, s: (b, 0, 0)),
            pl.BlockSpec((1, _ST, 3), lambda b, s: (b, s, 0)),
            pl.BlockSpec((1, _ST, _CO), lambda b, s: (b, s, 0)),
            pl.BlockSpec((1, 1, _RW, 1), lambda b, s: (b, s, 0, 0)),
            pl.BlockSpec((1, _ST, 1), lambda b, s: (b, s, 0)),
        ] + [pl.BlockSpec(None)] * 8,
        out_specs=[
            pl.BlockSpec((1, 1, _RW, 1), lambda b, s: (b, s, 0, 0)),
            pl.BlockSpec((1, 1, _RW, _CO), lambda b, s: (b, s, 0, 0)),
            pl.BlockSpec((1, 1, _NG, 2), lambda b, s: (b, s, 0, 0)),
        ],
        out_shape=[
            jax.ShapeDtypeStruct((_B, nst, _RW, 1), _f32),
            jax.ShapeDtypeStruct((_B, nst, _RW, _CO), _f32),
            jax.ShapeDtypeStruct((_B, nst, _NG, 2), _f32),
        ],
    )(fembT, xyz, new_xyz, new_feat, gidxF, count,
      p['fq_W'], row(p['fq_b']),
      p['fk_W'][:, :9], p['fk_W'][:, 9:], row(p['fk_b']),
      p['fv_W'][:, :9], p['fv_W'][:, 9:], row(p['fv_b']))

    outF = pl.pallas_call(
        _finish_kern,
        grid=(_B, nst),
        in_specs=[
            pl.BlockSpec((1, 1, _RW, _CO), lambda b, s: (b, s, 0, 0)),
            pl.BlockSpec((1, 1, _RW, 1), lambda b, s: (b, s, 0, 0)),
            pl.BlockSpec((1, nst, _NG, 2), lambda b, s: (b, 0, 0, 0)),
            pl.BlockSpec(None),
            pl.BlockSpec(None),
        ],
        out_specs=pl.BlockSpec((1, _ST, _CO), lambda b, s: (b, s, 0)),
        out_shape=jax.ShapeDtypeStruct((_B, _S, _CO), _f32),
    )(vconv, attn, stats, row(p['fv_g']), row(p['fv_be']))

    new_features = outF.transpose(0, 2, 1)
    return new_xyz, new_features
